# baseline (device time: 48830 ns/iter reference)
import jax
import jax.numpy as jnp
from jax import lax
from jax.experimental import pallas as pl
from jax.experimental.pallas import tpu as pltpu

N_DEV = 4
B_LOC = 2
SQ = 256
SKV = 256
HQ = 16
H_BLK = 4
DH = 64
D_MODEL = 512
D_BLK = H_BLK * DH


def kernel(x, Wq, K_ext, V_ext, Wo):
    def body(x_ref, wq_ref, k_hbm, v_hbm, wo_ref, out_ref,
             wq_comm, wo_comm, wq_send, wo_send,
             send_sems, recv_sems, copy_sems, k_vmem, v_vmem, acc, ctx_buf):
        my = lax.axis_index("i")

        kv_dmas = {}
        for s in (0, 1, 3, 2):
            origin = lax.rem(my + (N_DEV - s), N_DEV) if s else my
            dmas = []
            for b in range(B_LOC):
                bg = B_LOC * my + b
                for h in range(H_BLK):
                    hg = H_BLK * origin + h
                    kd = pltpu.make_async_copy(
                        k_hbm.at[bg, :, hg, :], k_vmem.at[s, b, h],
                        copy_sems.at[0, s],
                    )
                    vd = pltpu.make_async_copy(
                        v_hbm.at[bg, :, hg, :], v_vmem.at[s, b, h],
                        copy_sems.at[1, s],
                    )
                    kd.start()
                    vd.start()
                    dmas.append(kd)
                    dmas.append(vd)
            kv_dmas[s] = dmas

        wq_send[:] = wq_ref[:].astype(jnp.bfloat16)
        wo_send[:] = wo_ref[:].astype(jnp.bfloat16)

        barrier = pltpu.get_barrier_semaphore()
        for k in range(1, N_DEV):
            pl.semaphore_signal(
                barrier, inc=1,
                device_id=(lax.rem(my + k, N_DEV),),
                device_id_type=pl.DeviceIdType.MESH,
            )
        pl.semaphore_wait(barrier, N_DEV - 1)

        sends = []
        for k in range(1, N_DEV):
            dst = lax.rem(my + k, N_DEV)
            r_wq = pltpu.make_async_remote_copy(
                src_ref=wq_send,
                dst_ref=wq_comm.at[k - 1],
                send_sem=send_sems.at[2 * (k - 1)],
                recv_sem=recv_sems.at[2 * (k - 1)],
                device_id=(dst,),
                device_id_type=pl.DeviceIdType.MESH,
            )
            r_wo = pltpu.make_async_remote_copy(
                src_ref=wo_send,
                dst_ref=wo_comm.at[k - 1],
                send_sem=send_sems.at[2 * (k - 1) + 1],
                recv_sem=recv_sems.at[2 * (k - 1) + 1],
                device_id=(dst,),
                device_id_type=pl.DeviceIdType.MESH,
            )
            r_wq.start()
            r_wo.start()
            sends.append(r_wq)
            sends.append(r_wo)

        qi = lax.broadcasted_iota(jnp.int32, (SQ, SKV), 0)
        ki = lax.broadcasted_iota(jnp.int32, (SQ, SKV), 1)
        mask = (jnp.abs(qi - ki) <= 128) | (ki < 32) | (qi < 32)

        x2d = x_ref[:].reshape(B_LOC * SQ, D_MODEL).astype(jnp.bfloat16)

        def do_block(s, wq_blk, wo_blk):
            for d in kv_dmas[s]:
                d.wait()
            q = lax.dot_general(
                x2d, wq_blk, (((1,), (0,)), ((), ())),
                preferred_element_type=jnp.float32,
            )
            q_bf = q.astype(jnp.bfloat16)
            for b in range(B_LOC):
                for h in range(H_BLK):
                    qh = q_bf[b * SQ:(b + 1) * SQ, h * DH:(h + 1) * DH]
                    kh = k_vmem[s, b, h].astype(jnp.bfloat16)
                    vh = v_vmem[s, b, h].astype(jnp.bfloat16)
                    sc = lax.dot_general(
                        qh, kh, (((1,), (1,)), ((), ())),
                        preferred_element_type=jnp.float32,
                    ) * 0.125
                    sc = jnp.where(mask, sc, -1e9)
                    m = jnp.max(sc, axis=1, keepdims=True)
                    w = jnp.exp(sc - m)
                    w = w / jnp.sum(w, axis=1, keepdims=True)
                    ctx = lax.dot_general(
                        w.astype(jnp.bfloat16), vh, (((1,), (0,)), ((), ())),
                        preferred_element_type=jnp.float32,
                    )
                    ctx_buf[b * SQ:(b + 1) * SQ, h * DH:(h + 1) * DH] = (
                        ctx.astype(jnp.bfloat16)
                    )
            return lax.dot_general(
                ctx_buf[:], wo_blk, (((1,), (0,)), ((), ())),
                preferred_element_type=jnp.float32,
            )

        acc[:] = do_block(0, wq_send[:], wo_send[:])

        for k in (1, 3, 2):
            recv_wq = pltpu.make_async_remote_copy(
                src_ref=wq_send,
                dst_ref=wq_comm.at[k - 1],
                send_sem=send_sems.at[2 * (k - 1)],
                recv_sem=recv_sems.at[2 * (k - 1)],
                device_id=(my,),
                device_id_type=pl.DeviceIdType.MESH,
            )
            recv_wo = pltpu.make_async_remote_copy(
                src_ref=wo_send,
                dst_ref=wo_comm.at[k - 1],
                send_sem=send_sems.at[2 * (k - 1) + 1],
                recv_sem=recv_sems.at[2 * (k - 1) + 1],
                device_id=(my,),
                device_id_type=pl.DeviceIdType.MESH,
            )
            recv_wq.wait_recv()
            recv_wo.wait_recv()
            acc[:] += do_block(k, wq_comm[k - 1], wo_comm[k - 1])

        for r in sends:
            r.wait_send()

        out_ref[:] = acc[:].reshape(B_LOC, SQ, D_MODEL)

    return pl.pallas_call(
        body,
        out_shape=jax.ShapeDtypeStruct((B_LOC, SQ, D_MODEL), jnp.float32),
        in_specs=[
            pl.BlockSpec(memory_space=pltpu.VMEM),
            pl.BlockSpec(memory_space=pltpu.VMEM),
            pl.BlockSpec(memory_space=pl.ANY),
            pl.BlockSpec(memory_space=pl.ANY),
            pl.BlockSpec(memory_space=pltpu.VMEM),
        ],
        out_specs=pl.BlockSpec(memory_space=pltpu.VMEM),
        scratch_shapes=[
            pltpu.VMEM((N_DEV - 1, D_MODEL, D_BLK), jnp.bfloat16),
            pltpu.VMEM((N_DEV - 1, D_BLK, D_MODEL), jnp.bfloat16),
            pltpu.VMEM((D_MODEL, D_BLK), jnp.bfloat16),
            pltpu.VMEM((D_BLK, D_MODEL), jnp.bfloat16),
            pltpu.SemaphoreType.DMA((2 * (N_DEV - 1),)),
            pltpu.SemaphoreType.DMA((2 * (N_DEV - 1),)),
            pltpu.SemaphoreType.DMA((2, N_DEV)),
            pltpu.VMEM((N_DEV, B_LOC, H_BLK, SKV, DH), jnp.float32),
            pltpu.VMEM((N_DEV, B_LOC, H_BLK, SKV, DH), jnp.float32),
            pltpu.VMEM((B_LOC * SQ, D_MODEL), jnp.float32),
            pltpu.VMEM((B_LOC * SQ, D_BLK), jnp.bfloat16),
        ],
        compiler_params=pltpu.CompilerParams(collective_id=0),
    )(x, Wq, K_ext, V_ext, Wo)


# device time: 35092 ns/iter; 1.3915x vs baseline; 1.3915x over previous
import jax
import jax.numpy as jnp
from jax import lax
from jax.experimental import pallas as pl
from jax.experimental.pallas import tpu as pltpu

N_DEV = 4
B_LOC = 2
SQ = 256
SKV = 256
HQ = 16
H_BLK = 4
DH = 64
D_MODEL = 512
D_BLK = H_BLK * DH


def kernel(x, Wq, K_ext, V_ext, Wo):
    def body(x_ref, wq_ref, k_hbm, v_hbm, wo_ref, out_ref,
             wq_comm, wo_comm, wq_send, wo_send,
             send_sems, recv_sems, copy_sems, k_vmem, v_vmem, acc, ctx_buf):
        my = lax.axis_index("i")

        kv_dmas = {}
        for s in (0, 1, 3, 2):
            origin = lax.rem(my + (N_DEV - s), N_DEV) if s else my
            dmas = []
            for b in range(B_LOC):
                bg = B_LOC * my + b
                kd = pltpu.make_async_copy(
                    k_hbm.at[bg, :, pl.ds(origin * D_BLK, D_BLK)],
                    k_vmem.at[s, b],
                    copy_sems.at[0, s],
                )
                vd = pltpu.make_async_copy(
                    v_hbm.at[bg, :, pl.ds(origin * D_BLK, D_BLK)],
                    v_vmem.at[s, b],
                    copy_sems.at[1, s],
                )
                kd.start()
                vd.start()
                dmas.append(kd)
                dmas.append(vd)
            kv_dmas[s] = dmas

        wq_send[:] = wq_ref[:].astype(jnp.bfloat16)
        wo_send[:] = wo_ref[:].astype(jnp.bfloat16)

        barrier = pltpu.get_barrier_semaphore()
        for k in range(1, N_DEV):
            pl.semaphore_signal(
                barrier, inc=1,
                device_id=(lax.rem(my + k, N_DEV),),
                device_id_type=pl.DeviceIdType.MESH,
            )
        pl.semaphore_wait(barrier, N_DEV - 1)

        sends = []
        for k in range(1, N_DEV):
            dst = lax.rem(my + k, N_DEV)
            r_wq = pltpu.make_async_remote_copy(
                src_ref=wq_send,
                dst_ref=wq_comm.at[k - 1],
                send_sem=send_sems.at[2 * (k - 1)],
                recv_sem=recv_sems.at[2 * (k - 1)],
                device_id=(dst,),
                device_id_type=pl.DeviceIdType.MESH,
            )
            r_wo = pltpu.make_async_remote_copy(
                src_ref=wo_send,
                dst_ref=wo_comm.at[k - 1],
                send_sem=send_sems.at[2 * (k - 1) + 1],
                recv_sem=recv_sems.at[2 * (k - 1) + 1],
                device_id=(dst,),
                device_id_type=pl.DeviceIdType.MESH,
            )
            r_wq.start()
            r_wo.start()
            sends.append(r_wq)
            sends.append(r_wo)

        qi = lax.broadcasted_iota(jnp.int32, (SQ, SKV), 0)
        ki = lax.broadcasted_iota(jnp.int32, (SQ, SKV), 1)
        mask = (jnp.abs(qi - ki) <= 128) | (ki < 32) | (qi < 32)

        x2d = x_ref[:].reshape(B_LOC * SQ, D_MODEL).astype(jnp.bfloat16)

        def do_block(s, wq_blk, wo_blk):
            for d in kv_dmas[s]:
                d.wait()
            q = lax.dot_general(
                x2d, wq_blk, (((1,), (0,)), ((), ())),
                preferred_element_type=jnp.float32,
            )
            q_bf = q.astype(jnp.bfloat16)
            for b in range(B_LOC):
                k_bf = k_vmem[s, b].astype(jnp.bfloat16)
                v_bf = v_vmem[s, b].astype(jnp.bfloat16)
                for h in range(H_BLK):
                    qh = q_bf[b * SQ:(b + 1) * SQ, h * DH:(h + 1) * DH]
                    kh = k_bf[:, h * DH:(h + 1) * DH]
                    vh = v_bf[:, h * DH:(h + 1) * DH]
                    sc = lax.dot_general(
                        qh, kh, (((1,), (1,)), ((), ())),
                        preferred_element_type=jnp.float32,
                    ) * 0.125
                    sc = jnp.where(mask, sc, -1e9)
                    m = jnp.max(sc, axis=1, keepdims=True)
                    w = jnp.exp(sc - m)
                    w = w / jnp.sum(w, axis=1, keepdims=True)
                    ctx = lax.dot_general(
                        w.astype(jnp.bfloat16), vh, (((1,), (0,)), ((), ())),
                        preferred_element_type=jnp.float32,
                    )
                    ctx_buf[b * SQ:(b + 1) * SQ, h * DH:(h + 1) * DH] = (
                        ctx.astype(jnp.bfloat16)
                    )
            return lax.dot_general(
                ctx_buf[:], wo_blk, (((1,), (0,)), ((), ())),
                preferred_element_type=jnp.float32,
            )

        acc[:] = do_block(0, wq_send[:], wo_send[:])

        for k in (1, 3, 2):
            recv_wq = pltpu.make_async_remote_copy(
                src_ref=wq_send,
                dst_ref=wq_comm.at[k - 1],
                send_sem=send_sems.at[2 * (k - 1)],
                recv_sem=recv_sems.at[2 * (k - 1)],
                device_id=(my,),
                device_id_type=pl.DeviceIdType.MESH,
            )
            recv_wo = pltpu.make_async_remote_copy(
                src_ref=wo_send,
                dst_ref=wo_comm.at[k - 1],
                send_sem=send_sems.at[2 * (k - 1) + 1],
                recv_sem=recv_sems.at[2 * (k - 1) + 1],
                device_id=(my,),
                device_id_type=pl.DeviceIdType.MESH,
            )
            recv_wq.wait_recv()
            recv_wo.wait_recv()
            acc[:] += do_block(k, wq_comm[k - 1], wo_comm[k - 1])

        for r in sends:
            r.wait_send()

        out_ref[:] = acc[:].reshape(B_LOC, SQ, D_MODEL)

    return pl.pallas_call(
        body,
        out_shape=jax.ShapeDtypeStruct((B_LOC, SQ, D_MODEL), jnp.float32),
        in_specs=[
            pl.BlockSpec(memory_space=pltpu.VMEM),
            pl.BlockSpec(memory_space=pltpu.VMEM),
            pl.BlockSpec(memory_space=pl.ANY),
            pl.BlockSpec(memory_space=pl.ANY),
            pl.BlockSpec(memory_space=pltpu.VMEM),
        ],
        out_specs=pl.BlockSpec(memory_space=pltpu.VMEM),
        scratch_shapes=[
            pltpu.VMEM((N_DEV - 1, D_MODEL, D_BLK), jnp.bfloat16),
            pltpu.VMEM((N_DEV - 1, D_BLK, D_MODEL), jnp.bfloat16),
            pltpu.VMEM((D_MODEL, D_BLK), jnp.bfloat16),
            pltpu.VMEM((D_BLK, D_MODEL), jnp.bfloat16),
            pltpu.SemaphoreType.DMA((2 * (N_DEV - 1),)),
            pltpu.SemaphoreType.DMA((2 * (N_DEV - 1),)),
            pltpu.SemaphoreType.DMA((2, N_DEV)),
            pltpu.VMEM((N_DEV, B_LOC, SKV, D_BLK), jnp.float32),
            pltpu.VMEM((N_DEV, B_LOC, SKV, D_BLK), jnp.float32),
            pltpu.VMEM((B_LOC * SQ, D_MODEL), jnp.float32),
            pltpu.VMEM((B_LOC * SQ, D_BLK), jnp.bfloat16),
        ],
        compiler_params=pltpu.CompilerParams(collective_id=0),
    )(x, Wq,
      K_ext.reshape(K_ext.shape[0], SQ, HQ * DH),
      V_ext.reshape(V_ext.shape[0], SQ, HQ * DH),
      Wo)


# device time: 26599 ns/iter; 1.8358x vs baseline; 1.3193x over previous
import jax
import jax.numpy as jnp
from jax import lax
from jax.experimental import pallas as pl
from jax.experimental.pallas import tpu as pltpu

N_DEV = 4
B_LOC = 2
SQ = 256
SKV = 256
HQ = 16
H_BLK = 4
DH = 64
D_MODEL = 512
D_BLK = H_BLK * DH


def kernel(x, Wq, K_ext, V_ext, Wo):
    i = lax.axis_index("i")
    Kc = lax.dynamic_slice_in_dim(K_ext, B_LOC * i, B_LOC, axis=0).astype(
        jnp.bfloat16).reshape(B_LOC, SKV, HQ * DH)
    Vc = lax.dynamic_slice_in_dim(V_ext, B_LOC * i, B_LOC, axis=0).astype(
        jnp.bfloat16).reshape(B_LOC, SKV, HQ * DH)

    def body(x_ref, wq_ref, k_ref, v_ref, wo_ref, out_ref,
             wq_comm, wo_comm, wq_send, wo_send,
             send_sems, recv_sems, acc, ctx_buf):
        my = lax.axis_index("i")

        wq_send[:] = wq_ref[:].astype(jnp.bfloat16)
        wo_send[:] = wo_ref[:].astype(jnp.bfloat16)

        barrier = pltpu.get_barrier_semaphore()
        for k in range(1, N_DEV):
            pl.semaphore_signal(
                barrier, inc=1,
                device_id=(lax.rem(my + k, N_DEV),),
                device_id_type=pl.DeviceIdType.MESH,
            )
        pl.semaphore_wait(barrier, N_DEV - 1)

        sends = []
        for k in range(1, N_DEV):
            dst = lax.rem(my + k, N_DEV)
            r_wq = pltpu.make_async_remote_copy(
                src_ref=wq_send,
                dst_ref=wq_comm.at[k - 1],
                send_sem=send_sems.at[2 * (k - 1)],
                recv_sem=recv_sems.at[2 * (k - 1)],
                device_id=(dst,),
                device_id_type=pl.DeviceIdType.MESH,
            )
            r_wo = pltpu.make_async_remote_copy(
                src_ref=wo_send,
                dst_ref=wo_comm.at[k - 1],
                send_sem=send_sems.at[2 * (k - 1) + 1],
                recv_sem=recv_sems.at[2 * (k - 1) + 1],
                device_id=(dst,),
                device_id_type=pl.DeviceIdType.MESH,
            )
            r_wq.start()
            r_wo.start()
            sends.append(r_wq)
            sends.append(r_wo)

        qi = lax.broadcasted_iota(jnp.int32, (SQ, SKV), 0)
        ki = lax.broadcasted_iota(jnp.int32, (SQ, SKV), 1)
        mask = (jnp.abs(qi - ki) <= 128) | (ki < 32) | (qi < 32)

        x2d = x_ref[:].reshape(B_LOC * SQ, D_MODEL).astype(jnp.bfloat16)

        def do_block(origin, wq_blk, wo_blk):
            q = lax.dot_general(
                x2d, wq_blk, (((1,), (0,)), ((), ())),
                preferred_element_type=jnp.float32,
            )
            q_bf = q.astype(jnp.bfloat16)
            for b in range(B_LOC):
                for p in range(H_BLK // 2):
                    off = pl.multiple_of(origin * D_BLK, 128) + p * 128
                    k_pair = k_ref[b, :, pl.ds(off, 2 * DH)]
                    v_pair = v_ref[b, :, pl.ds(off, 2 * DH)]
                    for hh in range(2):
                        h = 2 * p + hh
                        qh = q_bf[b * SQ:(b + 1) * SQ, h * DH:(h + 1) * DH]
                        kh = k_pair[:, hh * DH:(hh + 1) * DH]
                        vh = v_pair[:, hh * DH:(hh + 1) * DH]
                        sc = lax.dot_general(
                            qh, kh, (((1,), (1,)), ((), ())),
                            preferred_element_type=jnp.float32,
                        ) * 0.125
                        sc = jnp.where(mask, sc, -1e9)
                        m = jnp.max(sc, axis=1, keepdims=True)
                        w = jnp.exp(sc - m)
                        w = w / jnp.sum(w, axis=1, keepdims=True)
                        ctx = lax.dot_general(
                            w.astype(jnp.bfloat16), vh, (((1,), (0,)), ((), ())),
                            preferred_element_type=jnp.float32,
                        )
                        ctx_buf[b * SQ:(b + 1) * SQ, h * DH:(h + 1) * DH] = (
                            ctx.astype(jnp.bfloat16)
                        )
            return lax.dot_general(
                ctx_buf[:], wo_blk, (((1,), (0,)), ((), ())),
                preferred_element_type=jnp.float32,
            )

        acc[:] = do_block(my, wq_send[:], wo_send[:])

        for k in (1, 3, 2):
            recv_wq = pltpu.make_async_remote_copy(
                src_ref=wq_send,
                dst_ref=wq_comm.at[k - 1],
                send_sem=send_sems.at[2 * (k - 1)],
                recv_sem=recv_sems.at[2 * (k - 1)],
                device_id=(my,),
                device_id_type=pl.DeviceIdType.MESH,
            )
            recv_wo = pltpu.make_async_remote_copy(
                src_ref=wo_send,
                dst_ref=wo_comm.at[k - 1],
                send_sem=send_sems.at[2 * (k - 1) + 1],
                recv_sem=recv_sems.at[2 * (k - 1) + 1],
                device_id=(my,),
                device_id_type=pl.DeviceIdType.MESH,
            )
            recv_wq.wait_recv()
            recv_wo.wait_recv()
            origin = lax.rem(my + (N_DEV - k), N_DEV)
            acc[:] += do_block(origin, wq_comm[k - 1], wo_comm[k - 1])

        for r in sends:
            r.wait_send()

        out_ref[:] = acc[:].reshape(B_LOC, SQ, D_MODEL)

    return pl.pallas_call(
        body,
        out_shape=jax.ShapeDtypeStruct((B_LOC, SQ, D_MODEL), jnp.float32),
        in_specs=[
            pl.BlockSpec(memory_space=pltpu.VMEM),
            pl.BlockSpec(memory_space=pltpu.VMEM),
            pl.BlockSpec(memory_space=pltpu.VMEM),
            pl.BlockSpec(memory_space=pltpu.VMEM),
            pl.BlockSpec(memory_space=pltpu.VMEM),
        ],
        out_specs=pl.BlockSpec(memory_space=pltpu.VMEM),
        scratch_shapes=[
            pltpu.VMEM((N_DEV - 1, D_MODEL, D_BLK), jnp.bfloat16),
            pltpu.VMEM((N_DEV - 1, D_BLK, D_MODEL), jnp.bfloat16),
            pltpu.VMEM((D_MODEL, D_BLK), jnp.bfloat16),
            pltpu.VMEM((D_BLK, D_MODEL), jnp.bfloat16),
            pltpu.SemaphoreType.DMA((2 * (N_DEV - 1),)),
            pltpu.SemaphoreType.DMA((2 * (N_DEV - 1),)),
            pltpu.VMEM((B_LOC * SQ, D_MODEL), jnp.float32),
            pltpu.VMEM((B_LOC * SQ, D_BLK), jnp.bfloat16),
        ],
        compiler_params=pltpu.CompilerParams(collective_id=0),
    )(x, Wq, Kc, Vc, Wo)
